# Initial kernel scaffold; baseline (speedup 1.0000x reference)
import jax
import jax.numpy as jnp
from jax.experimental import pallas as pl


def kernel(x, edge_index, edge_label, W_gnn, b_gnn, W_pred, b_pred):
    raise NotImplementedError("write your pallas kernel here")


# speedup vs baseline: 6.0359x; 6.0359x over previous
import functools

import jax
import jax.numpy as jnp
from jax import lax
from jax.experimental import pallas as pl
from jax.experimental.pallas import tpu as pltpu
from jax.experimental.pallas import tpu_sc as plsc

N = 10000
N2 = 10240
E = 320000
D = 128
NC = 2
NS = 16
NW = NC * NS
EPW = E // NW
C = 80
NCH = EPW // C
NST = 5
CPS = NCH // NST
RPT = N2 // NS
ZR = 128
L = 16

_MESH = dict(core_axis_name="c", subcore_axis_name="s")


def _sc_aggregate(x, src3, dst3):
    @functools.partial(
        pl.kernel,
        out_type=[
            jax.ShapeDtypeStruct((NC, N2, D), jnp.float32),
            jax.ShapeDtypeStruct((NW, 1, N2), jnp.float32),
        ],
        mesh=plsc.VectorSubcoreMesh(**_MESH),
        compiler_params=pltpu.CompilerParams(needs_layout_passes=False),
        scratch_types=[
            pltpu.VMEM((CPS, C), jnp.int32),
            pltpu.VMEM((CPS, C), jnp.int32),
            pltpu.VMEM((C, D), jnp.float32),
            pltpu.VMEM((N2,), jnp.float32),
            pltpu.VMEM_SHARED((N2, D), jnp.float32),
            pltpu.SemaphoreType.DMA,
        ],
    )
    def agg_kernel(x_hbm, src_hbm, dst_hbm, agg_out, deg_out,
                   src_v, dst_v, rows_v, deg_v, agg_sh, sem):
        c = lax.axis_index("c")
        s = lax.axis_index("s")
        wid = s * NC + c

        zero16 = jnp.zeros((L,), jnp.float32)

        def zrow(i, carry):
            for k in range(D // L):
                rows_v[i, pl.ds(k * L, L)] = zero16
            return carry

        lax.fori_loop(0, C, zrow, 0)

        def zdeg(i, carry):
            deg_v[pl.ds(i * L, L)] = zero16
            return carry

        lax.fori_loop(0, N2 // L, zdeg, 0)

        for t in range(RPT // C):
            pltpu.sync_copy(rows_v, agg_sh.at[pl.ds(s * RPT + t * C, C)])
        plsc.subcore_barrier()

        ones16 = jnp.ones((L,), jnp.float32)
        for st in range(NST):
            pltpu.sync_copy(src_hbm.at[wid, st], src_v)
            pltpu.sync_copy(dst_hbm.at[wid, st], dst_v)

            def chunk(j, carry):
                pltpu.async_copy(x_hbm.at[src_v.at[j]], rows_v, sem).wait()
                pltpu.sync_copy(rows_v, agg_sh.at[dst_v.at[j]], add=True)
                for k in range(C // L):
                    idx = dst_v[j, pl.ds(k * L, L)]
                    plsc.addupdate_scatter(deg_v, [idx], ones16)
                return carry

            lax.fori_loop(0, CPS, chunk, 0)
        plsc.subcore_barrier()

        for t in range(RPT // C):
            r0 = s * RPT + t * C
            pltpu.sync_copy(agg_sh.at[pl.ds(r0, C)],
                            agg_out.at[c, pl.ds(r0, C)])
        pltpu.sync_copy(deg_v, deg_out.at[wid, 0])

    return agg_kernel(x, src3, dst3)


def _tc_dense(agg2, deg_t, w1, b1, w2, b2):
    def body(a_ref, d_ref, w1_ref, b1_ref, w2_ref, b2_ref, o_ref):
        a = a_ref[0] + a_ref[1]
        deg = jnp.maximum(jnp.sum(d_ref[...], axis=1, keepdims=True), 1.0)
        a = a * (1.0 / deg)
        h = jnp.dot(a, w1_ref[...], preferred_element_type=jnp.float32)
        h = jnp.maximum(h + b1_ref[...], 0.0)
        o_ref[...] = (jnp.dot(h, w2_ref[...], preferred_element_type=jnp.float32)
                      + b2_ref[...])

    return pl.pallas_call(
        body,
        out_shape=jax.ShapeDtypeStruct((N2, D), jnp.float32),
    )(agg2, deg_t, w1, b1.reshape(1, D), w2, b2.reshape(1, D))


def _sc_edge_scores(ne, src3, dst3):
    @functools.partial(
        pl.kernel,
        out_type=jax.ShapeDtypeStruct((NW, 1, EPW), jnp.float32),
        mesh=plsc.VectorSubcoreMesh(**_MESH),
        compiler_params=pltpu.CompilerParams(needs_layout_passes=False),
        scratch_types=[
            pltpu.VMEM((CPS, C), jnp.int32),
            pltpu.VMEM((CPS, C), jnp.int32),
            pltpu.VMEM((C, D), jnp.float32),
            pltpu.VMEM((C, D), jnp.float32),
            pltpu.VMEM((EPW,), jnp.float32),
            pltpu.SemaphoreType.DMA,
            pltpu.SemaphoreType.DMA,
        ],
    )
    def score_kernel(ne_hbm, src_hbm, dst_hbm, pred_out,
                     src_v, dst_v, a_v, b_v, pred_v, sem_a, sem_b):
        c = lax.axis_index("c")
        s = lax.axis_index("s")
        wid = s * NC + c
        last_lane = lax.iota(jnp.int32, L) == (L - 1)

        for st in range(NST):
            pltpu.sync_copy(src_hbm.at[wid, st], src_v)
            pltpu.sync_copy(dst_hbm.at[wid, st], dst_v)

            def chunk(j, carry):
                cp_a = pltpu.async_copy(ne_hbm.at[src_v.at[j]], a_v, sem_a)
                cp_b = pltpu.async_copy(ne_hbm.at[dst_v.at[j]], b_v, sem_b)
                cp_a.wait()
                cp_b.wait()
                base = (st * CPS + j) * C

                def edge(e, inner):
                    acc = a_v[e, pl.ds(0, L)] * b_v[e, pl.ds(0, L)]
                    for k in range(1, D // L):
                        acc = acc + a_v[e, pl.ds(k * L, L)] * b_v[e, pl.ds(k * L, L)]
                    cum = plsc.cumsum(acc)
                    idx = jnp.full((L,), 0, jnp.int32) + (base + e)
                    plsc.store_scatter(pred_v, [idx], cum, mask=last_lane)
                    return inner

                lax.fori_loop(0, C, edge, 0)
                return carry

            lax.fori_loop(0, CPS, chunk, 0)
        pltpu.sync_copy(pred_v, pred_out.at[wid, 0])

    return score_kernel(ne, src3, dst3)


def _tc_loss(pred2d, y2d):
    def body(p_ref, y_ref, o_ref):
        p = p_ref[...]
        y = y_ref[...].astype(jnp.float32)
        t = (jnp.maximum(p, 0.0) - p * y
             + jnp.log(1.0 + jnp.exp(-jnp.abs(p))))
        o_ref[0, 0] = jnp.sum(t)

    return pl.pallas_call(
        body,
        out_shape=jax.ShapeDtypeStruct((1, 1), jnp.float32),
        out_specs=pl.BlockSpec(memory_space=pltpu.SMEM),
    )(pred2d, y2d)


def kernel(x, edge_index, edge_label, W_gnn, b_gnn, W_pred, b_pred):
    src3 = edge_index[0].reshape(NW, NST, CPS, C)
    dst3 = edge_index[1].reshape(NW, NST, CPS, C)
    agg2, deg = _sc_aggregate(x, src3, dst3)
    ne = _tc_dense(agg2, deg.reshape(NW, N2).T, W_gnn, b_gnn, W_pred, b_pred)
    pred = _sc_edge_scores(ne, src3, dst3)
    total = _tc_loss(pred.reshape(E // D, D), edge_label.reshape(E // D, D))
    return total[0, 0] / E


# speedup vs baseline: 11.1923x; 1.8543x over previous
import functools

import jax
import jax.numpy as jnp
from jax import lax
from jax.experimental import pallas as pl
from jax.experimental.pallas import tpu as pltpu
from jax.experimental.pallas import tpu_sc as plsc

N = 10000
N2 = 10240
E = 320000
D = 128
NC = 2
NS = 16
NW = NC * NS
EPW = E // NW
C = 80
NCH = EPW // C
NST = 5
CPS = NCH // NST
RPT = N2 // NS
ZR = 128
L = 16

_MESH = dict(core_axis_name="c", subcore_axis_name="s")


def _sc_aggregate(x, src3, dst3):
    @functools.partial(
        pl.kernel,
        out_type=[
            jax.ShapeDtypeStruct((NC, N2, D), jnp.float32),
            jax.ShapeDtypeStruct((NW, 1, N2), jnp.float32),
        ],
        mesh=plsc.VectorSubcoreMesh(**_MESH),
        compiler_params=pltpu.CompilerParams(needs_layout_passes=False),
        scratch_types=[
            pltpu.VMEM((CPS, C), jnp.int32),
            pltpu.VMEM((CPS, C), jnp.int32),
            pltpu.VMEM((C, D), jnp.float32),
            pltpu.VMEM((C, D), jnp.float32),
            pltpu.VMEM((N2,), jnp.float32),
            pltpu.VMEM_SHARED((N2, D), jnp.float32),
            pltpu.SemaphoreType.DMA,
            pltpu.SemaphoreType.DMA,
        ],
    )
    def agg_kernel(x_hbm, src_hbm, dst_hbm, agg_out, deg_out,
                   src_v, dst_v, rows0_v, rows1_v, deg_v, agg_sh, sem0, sem1):
        c = lax.axis_index("c")
        s = lax.axis_index("s")
        wid = s * NC + c

        zero16 = jnp.zeros((L,), jnp.float32)

        def zrow(i, carry):
            for k in range(D // L):
                rows0_v[i, pl.ds(k * L, L)] = zero16
            return carry

        lax.fori_loop(0, C, zrow, 0)

        def zdeg(i, carry):
            deg_v[pl.ds(i * L, L)] = zero16
            return carry

        lax.fori_loop(0, N2 // L, zdeg, 0)

        for t in range(RPT // C):
            pltpu.sync_copy(rows0_v, agg_sh.at[pl.ds(s * RPT + t * C, C)])
        plsc.subcore_barrier()

        ones16 = jnp.ones((L,), jnp.float32)

        def count_deg(j):
            for k in range(C // L):
                idx = dst_v[j, pl.ds(k * L, L)]
                plsc.addupdate_scatter(deg_v, [idx], ones16)

        for st in range(NST):
            pltpu.sync_copy(src_hbm.at[wid, st], src_v)
            pltpu.sync_copy(dst_hbm.at[wid, st], dst_v)
            pltpu.async_copy(x_hbm.at[src_v.at[0]], rows0_v, sem0)
            pltpu.async_copy(x_hbm.at[src_v.at[1]], rows1_v, sem1)

            def pair(p, carry):
                j0 = 2 * p
                pltpu.make_async_copy(x_hbm.at[src_v.at[j0]],
                                      rows0_v, sem0).wait()
                count_deg(j0)
                pltpu.sync_copy(rows0_v, agg_sh.at[dst_v.at[j0]], add=True)
                pltpu.async_copy(x_hbm.at[src_v.at[j0 + 2]], rows0_v, sem0)

                j1 = j0 + 1
                pltpu.make_async_copy(x_hbm.at[src_v.at[j1]],
                                      rows1_v, sem1).wait()
                count_deg(j1)
                pltpu.sync_copy(rows1_v, agg_sh.at[dst_v.at[j1]], add=True)

                @pl.when(j1 + 2 < CPS)
                def _():
                    pltpu.async_copy(x_hbm.at[src_v.at[j1 + 2]],
                                     rows1_v, sem1)

                return carry

            lax.fori_loop(0, CPS // 2, pair, 0)
            jt = CPS - 1
            pltpu.make_async_copy(x_hbm.at[src_v.at[jt]], rows0_v, sem0).wait()
            count_deg(jt)
            pltpu.sync_copy(rows0_v, agg_sh.at[dst_v.at[jt]], add=True)
        plsc.subcore_barrier()

        for t in range(RPT // C):
            r0 = s * RPT + t * C
            pltpu.sync_copy(agg_sh.at[pl.ds(r0, C)],
                            agg_out.at[c, pl.ds(r0, C)])
        pltpu.sync_copy(deg_v, deg_out.at[wid, 0])

    return agg_kernel(x, src3, dst3)


def _tc_dense(agg2, deg_t, w1, b1, w2, b2):
    def body(a_ref, d_ref, w1_ref, b1_ref, w2_ref, b2_ref, o_ref):
        a = a_ref[0] + a_ref[1]
        deg = jnp.maximum(jnp.sum(d_ref[...], axis=1, keepdims=True), 1.0)
        a = a * (1.0 / deg)
        h = jnp.dot(a, w1_ref[...], preferred_element_type=jnp.float32)
        h = jnp.maximum(h + b1_ref[...], 0.0)
        o_ref[...] = (jnp.dot(h, w2_ref[...], preferred_element_type=jnp.float32)
                      + b2_ref[...])

    return pl.pallas_call(
        body,
        out_shape=jax.ShapeDtypeStruct((N2, D), jnp.float32),
    )(agg2, deg_t, w1, b1.reshape(1, D), w2, b2.reshape(1, D))


def _sc_edge_scores(ne, src3, dst3):
    @functools.partial(
        pl.kernel,
        out_type=jax.ShapeDtypeStruct((NW, 1, EPW), jnp.float32),
        mesh=plsc.VectorSubcoreMesh(**_MESH),
        compiler_params=pltpu.CompilerParams(needs_layout_passes=False),
        scratch_types=[
            pltpu.VMEM((CPS, C), jnp.int32),
            pltpu.VMEM((CPS, C), jnp.int32),
            pltpu.VMEM((C, D), jnp.float32),
            pltpu.VMEM((C, D), jnp.float32),
            pltpu.VMEM((C, D), jnp.float32),
            pltpu.VMEM((C, D), jnp.float32),
            pltpu.VMEM((EPW,), jnp.float32),
            pltpu.SemaphoreType.DMA,
            pltpu.SemaphoreType.DMA,
            pltpu.SemaphoreType.DMA,
            pltpu.SemaphoreType.DMA,
        ],
    )
    def score_kernel(ne_hbm, src_hbm, dst_hbm, pred_out,
                     src_v, dst_v, a0_v, b0_v, a1_v, b1_v, pred_v,
                     sem_a0, sem_b0, sem_a1, sem_b1):
        c = lax.axis_index("c")
        s = lax.axis_index("s")
        wid = s * NC + c
        last_lane = lax.iota(jnp.int32, L) == (L - 1)

        def score_chunk(st, j, a_v, b_v):
            base = (st * CPS + j) * C

            @plsc.parallel_loop(0, C, unroll=4)
            def edge(e):
                acc = a_v[e, pl.ds(0, L)] * b_v[e, pl.ds(0, L)]
                for k in range(1, D // L):
                    acc = acc + a_v[e, pl.ds(k * L, L)] * b_v[e, pl.ds(k * L, L)]
                cum = plsc.cumsum(acc)
                idx = jnp.full((L,), 0, jnp.int32) + (base + e)
                plsc.store_scatter(pred_v, [idx], cum, mask=last_lane)

        for st in range(NST):
            pltpu.sync_copy(src_hbm.at[wid, st], src_v)
            pltpu.sync_copy(dst_hbm.at[wid, st], dst_v)
            pltpu.async_copy(ne_hbm.at[src_v.at[0]], a0_v, sem_a0)
            pltpu.async_copy(ne_hbm.at[dst_v.at[0]], b0_v, sem_b0)
            pltpu.async_copy(ne_hbm.at[src_v.at[1]], a1_v, sem_a1)
            pltpu.async_copy(ne_hbm.at[dst_v.at[1]], b1_v, sem_b1)

            def pair(p, carry):
                j0 = 2 * p
                pltpu.make_async_copy(ne_hbm.at[src_v.at[j0]],
                                      a0_v, sem_a0).wait()
                pltpu.make_async_copy(ne_hbm.at[dst_v.at[j0]],
                                      b0_v, sem_b0).wait()
                score_chunk(st, j0, a0_v, b0_v)
                pltpu.async_copy(ne_hbm.at[src_v.at[j0 + 2]], a0_v, sem_a0)
                pltpu.async_copy(ne_hbm.at[dst_v.at[j0 + 2]], b0_v, sem_b0)

                j1 = j0 + 1
                pltpu.make_async_copy(ne_hbm.at[src_v.at[j1]],
                                      a1_v, sem_a1).wait()
                pltpu.make_async_copy(ne_hbm.at[dst_v.at[j1]],
                                      b1_v, sem_b1).wait()
                score_chunk(st, j1, a1_v, b1_v)

                @pl.when(j1 + 2 < CPS)
                def _():
                    pltpu.async_copy(ne_hbm.at[src_v.at[j1 + 2]], a1_v, sem_a1)
                    pltpu.async_copy(ne_hbm.at[dst_v.at[j1 + 2]], b1_v, sem_b1)

                return carry

            lax.fori_loop(0, CPS // 2, pair, 0)
            jt = CPS - 1
            pltpu.make_async_copy(ne_hbm.at[src_v.at[jt]], a0_v, sem_a0).wait()
            pltpu.make_async_copy(ne_hbm.at[dst_v.at[jt]], b0_v, sem_b0).wait()
            score_chunk(st, jt, a0_v, b0_v)
        pltpu.sync_copy(pred_v, pred_out.at[wid, 0])

    return score_kernel(ne, src3, dst3)


def _tc_loss(pred2d, y2d):
    def body(p_ref, y_ref, o_ref):
        p = p_ref[...]
        y = y_ref[...].astype(jnp.float32)
        t = (jnp.maximum(p, 0.0) - p * y
             + jnp.log(1.0 + jnp.exp(-jnp.abs(p))))
        o_ref[0, 0] = jnp.sum(t)

    return pl.pallas_call(
        body,
        out_shape=jax.ShapeDtypeStruct((1, 1), jnp.float32),
        out_specs=pl.BlockSpec(memory_space=pltpu.SMEM),
    )(pred2d, y2d)


def kernel(x, edge_index, edge_label, W_gnn, b_gnn, W_pred, b_pred):
    src3 = edge_index[0].reshape(NW, NST, CPS, C)
    dst3 = edge_index[1].reshape(NW, NST, CPS, C)
    agg2, deg = _sc_aggregate(x, src3, dst3)
    ne = _tc_dense(agg2, deg.reshape(NW, N2).T, W_gnn, b_gnn, W_pred, b_pred)
    pred = _sc_edge_scores(ne, src3, dst3)
    total = _tc_loss(pred.reshape(E // D, D), edge_label.reshape(E // D, D))
    return total[0, 0] / E


# speedup vs baseline: 13.6761x; 1.2219x over previous
import functools

import jax
import jax.numpy as jnp
from jax import lax
from jax.experimental import pallas as pl
from jax.experimental.pallas import tpu as pltpu
from jax.experimental.pallas import tpu_sc as plsc

N = 10000
N2 = 10240
E = 320000
D = 128
NC = 2
NS = 16
NW = NC * NS
EPW = E // NW
C = 80
NCH = EPW // C
NST = 5
CPS = NCH // NST
RPT = N2 // NS
ZR = 128
L = 16

_MESH = dict(core_axis_name="c", subcore_axis_name="s")


def _sc_aggregate(x, src3, dst3):
    @functools.partial(
        pl.kernel,
        out_type=[
            jax.ShapeDtypeStruct((NC, N2, D), jnp.float32),
            jax.ShapeDtypeStruct((NW, 1, N2), jnp.float32),
        ],
        mesh=plsc.VectorSubcoreMesh(**_MESH),
        compiler_params=pltpu.CompilerParams(needs_layout_passes=False),
        scratch_types=[
            pltpu.VMEM((CPS, C), jnp.int32),
            pltpu.VMEM((CPS, C), jnp.int32),
            pltpu.VMEM((C, D), jnp.float32),
            pltpu.VMEM((C, D), jnp.float32),
            pltpu.VMEM((N2,), jnp.float32),
            pltpu.VMEM_SHARED((N2, D), jnp.float32),
            pltpu.SemaphoreType.DMA,
            pltpu.SemaphoreType.DMA,
        ],
    )
    def agg_kernel(x_hbm, src_hbm, dst_hbm, agg_out, deg_out,
                   src_v, dst_v, rows0_v, rows1_v, deg_v, agg_sh, sem0, sem1):
        c = lax.axis_index("c")
        s = lax.axis_index("s")
        wid = s * NC + c

        zero16 = jnp.zeros((L,), jnp.float32)

        def zrow(i, carry):
            for k in range(D // L):
                rows0_v[i, pl.ds(k * L, L)] = zero16
            return carry

        lax.fori_loop(0, C, zrow, 0)

        def zdeg(i, carry):
            deg_v[pl.ds(i * L, L)] = zero16
            return carry

        lax.fori_loop(0, N2 // L, zdeg, 0)

        for t in range(RPT // C):
            pltpu.sync_copy(rows0_v, agg_sh.at[pl.ds(s * RPT + t * C, C)])
        plsc.subcore_barrier()

        ones16 = jnp.ones((L,), jnp.float32)

        def count_deg(j):
            for k in range(C // L):
                idx = dst_v[j, pl.ds(k * L, L)]
                plsc.addupdate_scatter(deg_v, [idx], ones16)

        for st in range(NST):
            pltpu.sync_copy(src_hbm.at[wid, st], src_v)
            pltpu.sync_copy(dst_hbm.at[wid, st], dst_v)
            pltpu.async_copy(x_hbm.at[src_v.at[0]], rows0_v, sem0)
            pltpu.async_copy(x_hbm.at[src_v.at[1]], rows1_v, sem1)

            def pair(p, carry):
                j0 = 2 * p
                pltpu.make_async_copy(x_hbm.at[src_v.at[j0]],
                                      rows0_v, sem0).wait()
                count_deg(j0)
                pltpu.sync_copy(rows0_v, agg_sh.at[dst_v.at[j0]], add=True)
                pltpu.async_copy(x_hbm.at[src_v.at[j0 + 2]], rows0_v, sem0)

                j1 = j0 + 1
                pltpu.make_async_copy(x_hbm.at[src_v.at[j1]],
                                      rows1_v, sem1).wait()
                count_deg(j1)
                pltpu.sync_copy(rows1_v, agg_sh.at[dst_v.at[j1]], add=True)

                @pl.when(j1 + 2 < CPS)
                def _():
                    pltpu.async_copy(x_hbm.at[src_v.at[j1 + 2]],
                                     rows1_v, sem1)

                return carry

            lax.fori_loop(0, CPS // 2, pair, 0)
            jt = CPS - 1
            pltpu.make_async_copy(x_hbm.at[src_v.at[jt]], rows0_v, sem0).wait()
            count_deg(jt)
            pltpu.sync_copy(rows0_v, agg_sh.at[dst_v.at[jt]], add=True)
        plsc.subcore_barrier()

        for t in range(RPT // C):
            r0 = s * RPT + t * C
            pltpu.sync_copy(agg_sh.at[pl.ds(r0, C)],
                            agg_out.at[c, pl.ds(r0, C)])
        pltpu.sync_copy(deg_v, deg_out.at[wid, 0])

    return agg_kernel(x, src3, dst3)


def _tc_dense(agg2, deg_t, w1, b1, w2, b2):
    def body(a_ref, d_ref, w1_ref, b1_ref, w2_ref, b2_ref, o_ref):
        a = a_ref[0] + a_ref[1]
        deg = jnp.maximum(jnp.sum(d_ref[...], axis=1, keepdims=True), 1.0)
        a = a * (1.0 / deg)
        h = jnp.dot(a, w1_ref[...], preferred_element_type=jnp.float32)
        h = jnp.maximum(h + b1_ref[...], 0.0)
        ne = (jnp.dot(h, w2_ref[...], preferred_element_type=jnp.float32)
              + b2_ref[...])
        ne_bf = ne.astype(jnp.bfloat16)
        lo = lax.convert_element_type(
            lax.bitcast_convert_type(ne_bf[:, :D // 2], jnp.uint16),
            jnp.uint32)
        hi = lax.convert_element_type(
            lax.bitcast_convert_type(ne_bf[:, D // 2:], jnp.uint16),
            jnp.uint32)
        o_ref[...] = lax.bitcast_convert_type(lo | (hi << 16), jnp.int32)

    return pl.pallas_call(
        body,
        out_shape=jax.ShapeDtypeStruct((N2, D // 2), jnp.int32),
    )(agg2, deg_t, w1, b1.reshape(1, D), w2, b2.reshape(1, D))


def _sc_edge_scores(ne, src3, dst3):
    @functools.partial(
        pl.kernel,
        out_type=jax.ShapeDtypeStruct((NW, 1, EPW), jnp.float32),
        mesh=plsc.VectorSubcoreMesh(**_MESH),
        compiler_params=pltpu.CompilerParams(needs_layout_passes=False,
                                             use_tc_tiling_on_sc=False),
        scratch_types=[
            pltpu.VMEM((CPS, C), jnp.int32),
            pltpu.VMEM((CPS, C), jnp.int32),
            pltpu.VMEM((C, D // 2), jnp.int32),
            pltpu.VMEM((C, D // 2), jnp.int32),
            pltpu.VMEM((C, D // 2), jnp.int32),
            pltpu.VMEM((C, D // 2), jnp.int32),
            pltpu.VMEM((EPW,), jnp.float32),
            pltpu.SemaphoreType.DMA,
            pltpu.SemaphoreType.DMA,
            pltpu.SemaphoreType.DMA,
            pltpu.SemaphoreType.DMA,
        ],
    )
    def score_kernel(ne_hbm, src_hbm, dst_hbm, pred_out,
                     src_v, dst_v, a0_v, b0_v, a1_v, b1_v, pred_v,
                     sem_a0, sem_b0, sem_a1, sem_b1):
        c = lax.axis_index("c")
        s = lax.axis_index("s")
        wid = s * NC + c
        last_lane = lax.iota(jnp.int32, L) == (L - 1)

        def score_chunk(st, j, a_v, b_v):
            base = (st * CPS + j) * C

            @plsc.parallel_loop(0, C, unroll=4)
            def edge(e):
                acc = jnp.zeros((L,), jnp.float32)
                for k in range(D // (2 * L)):
                    pa = plsc.bitcast(a_v[e, pl.ds(k * L, L)], jnp.bfloat16)
                    pb = plsc.bitcast(b_v[e, pl.ds(k * L, L)], jnp.bfloat16)
                    prod = pa * pb
                    u0, u1 = plsc.unpack(prod, format=plsc.PackFormat.INTERLEAVED)
                    acc = acc + u0 + u1
                cum = plsc.cumsum(acc)
                idx = jnp.full((L,), 0, jnp.int32) + (base + e)
                plsc.store_scatter(pred_v, [idx], cum, mask=last_lane)

        for st in range(NST):
            pltpu.sync_copy(src_hbm.at[wid, st], src_v)
            pltpu.sync_copy(dst_hbm.at[wid, st], dst_v)
            pltpu.async_copy(ne_hbm.at[src_v.at[0]], a0_v, sem_a0)
            pltpu.async_copy(ne_hbm.at[dst_v.at[0]], b0_v, sem_b0)
            pltpu.async_copy(ne_hbm.at[src_v.at[1]], a1_v, sem_a1)
            pltpu.async_copy(ne_hbm.at[dst_v.at[1]], b1_v, sem_b1)

            def pair(p, carry):
                j0 = 2 * p
                pltpu.make_async_copy(ne_hbm.at[src_v.at[j0]],
                                      a0_v, sem_a0).wait()
                pltpu.make_async_copy(ne_hbm.at[dst_v.at[j0]],
                                      b0_v, sem_b0).wait()
                score_chunk(st, j0, a0_v, b0_v)
                pltpu.async_copy(ne_hbm.at[src_v.at[j0 + 2]], a0_v, sem_a0)
                pltpu.async_copy(ne_hbm.at[dst_v.at[j0 + 2]], b0_v, sem_b0)

                j1 = j0 + 1
                pltpu.make_async_copy(ne_hbm.at[src_v.at[j1]],
                                      a1_v, sem_a1).wait()
                pltpu.make_async_copy(ne_hbm.at[dst_v.at[j1]],
                                      b1_v, sem_b1).wait()
                score_chunk(st, j1, a1_v, b1_v)

                @pl.when(j1 + 2 < CPS)
                def _():
                    pltpu.async_copy(ne_hbm.at[src_v.at[j1 + 2]], a1_v, sem_a1)
                    pltpu.async_copy(ne_hbm.at[dst_v.at[j1 + 2]], b1_v, sem_b1)

                return carry

            lax.fori_loop(0, CPS // 2, pair, 0)
            jt = CPS - 1
            pltpu.make_async_copy(ne_hbm.at[src_v.at[jt]], a0_v, sem_a0).wait()
            pltpu.make_async_copy(ne_hbm.at[dst_v.at[jt]], b0_v, sem_b0).wait()
            score_chunk(st, jt, a0_v, b0_v)
        pltpu.sync_copy(pred_v, pred_out.at[wid, 0])

    return score_kernel(ne, src3, dst3)


def _tc_loss(pred2d, y2d):
    def body(p_ref, y_ref, o_ref):
        p = p_ref[...]
        y = y_ref[...].astype(jnp.float32)
        t = (jnp.maximum(p, 0.0) - p * y
             + jnp.log(1.0 + jnp.exp(-jnp.abs(p))))
        o_ref[0, 0] = jnp.sum(t)

    return pl.pallas_call(
        body,
        out_shape=jax.ShapeDtypeStruct((1, 1), jnp.float32),
        out_specs=pl.BlockSpec(memory_space=pltpu.SMEM),
    )(pred2d, y2d)


def kernel(x, edge_index, edge_label, W_gnn, b_gnn, W_pred, b_pred):
    src3 = edge_index[0].reshape(NW, NST, CPS, C)
    dst3 = edge_index[1].reshape(NW, NST, CPS, C)
    agg2, deg = _sc_aggregate(x, src3, dst3)
    ne = _tc_dense(agg2, deg.reshape(NW, N2).T, W_gnn, b_gnn, W_pred, b_pred)
    pred = _sc_edge_scores(ne, src3, dst3)
    total = _tc_loss(pred.reshape(E // D, D), edge_label.reshape(E // D, D))
    return total[0, 0] / E
